# baseline (speedup 1.0000x reference)
import functools

import jax
import jax.numpy as jnp
from jax import lax
from jax.experimental import pallas as pl
from jax.experimental.pallas import tpu as pltpu
from jax.experimental.pallas import tpu_sc as plsc

B = 16384
S = 20
D = 64
V = 1000000
V_MAIN = 999424
V_CUT = 999936

NT = 16
BT = B // NT
TOK = S * BT

W = 8192
WIN_SHIFT = 13
OFF_MASK = W - 1
NWIN = 123
CSPLIT = W // NT

G = 4
NSWEEP = 8
CH = 1024
NVR = CH // 16
BKT_CAP = TOK + NWIN * 15 + CH + 16
SENT = 0x7FFFFFFF

_mesh = plsc.VectorSubcoreMesh(core_axis_name="c", subcore_axis_name="s")


def _ranks(srtw, iota):
    nxt = jnp.take_along_axis(srtw, jnp.minimum(iota + 1, 15), axis=0,
                              mode="promise_in_bounds")
    prv = jnp.take_along_axis(srtw, jnp.maximum(iota - 1, 0), axis=0,
                              mode="promise_in_bounds")
    end = (srtw != nxt) | (iota == 15)
    newg = (srtw != prv) | (iota == 0)
    start = plsc.cummax(jnp.where(newg, iota, 0))
    rank = iota - start
    return rank, end


@functools.partial(
    pl.kernel,
    mesh=_mesh,
    out_type=jax.ShapeDtypeStruct((S * D * B,), jnp.float32),
    scratch_types=[
        pltpu.VMEM((BKT_CAP,), jnp.int32),
        pltpu.VMEM((TOK,), jnp.float32),
        pltpu.VMEM((TOK,), jnp.float32),
        pltpu.VMEM((TOK,), jnp.float32),
        pltpu.VMEM((TOK,), jnp.float32),
        pltpu.VMEM((CH,), jnp.int32),
        pltpu.VMEM((CH,), jnp.int32),
        pltpu.VMEM((CH,), jnp.int32),
        pltpu.VMEM((CH,), jnp.int32),
        pltpu.VMEM((CH,), jnp.float32),
        pltpu.VMEM((CH,), jnp.float32),
        pltpu.VMEM((CH,), jnp.float32),
        pltpu.VMEM((CH,), jnp.float32),
        pltpu.VMEM((8, CSPLIT), jnp.float32),
        pltpu.VMEM((128,), jnp.int32),
        pltpu.VMEM((144,), jnp.int32),
        pltpu.VMEM((128,), jnp.int32),
        pltpu.VMEM_SHARED((16 * W,), jnp.float32),
        pltpu.SemaphoreType.DMA,
        pltpu.SemaphoreType.DMA,
        pltpu.SemaphoreType.DMA,
        pltpu.SemaphoreType.DMA,
    ],
    compiler_params=pltpu.CompilerParams(needs_layout_passes=False),
)
def _embed_kernel(tok_hbm, wt_hbm, wtail_hbm, out_hbm,
                  bkt, outv0, outv1, outv2, outv3,
                  idxb0, idxb1, idxb2, idxb3, gath0, gath1, gath2, gath3,
                  bounce, hist, seg, base, spflat,
                  sem_hbm, sem_spread, sem_gath, sem_flush):
    outv = [outv0, outv1, outv2, outv3]
    idxb = [idxb0, idxb1, idxb2, idxb3]
    gath = [gath0, gath1, gath2, gath3]
    c = lax.axis_index("c")
    s = lax.axis_index("s")
    b0 = s * BT
    iota = lax.iota(jnp.int32, 16)
    zero16 = jnp.zeros((16,), jnp.int32)
    sent16 = jnp.full((16,), SENT, jnp.int32)

    for i in range(8):
        hist[pl.ds(i * 16, 16)] = zero16

    def _init_bkt(i, carry):
        bkt[pl.ds(i * 16, 16)] = sent16
        return carry
    lax.fori_loop(0, BKT_CAP // 16, _init_bkt, 0)

    def _p1_chunk(ci, carry):
        pltpu.sync_copy(tok_hbm.at[pl.ds(ci * B + b0, BT)], idxb[0])

        def _p1_vec(i, carry2):
            t = idxb[0][pl.ds(i * 16, 16)]
            win = jnp.right_shift(t, WIN_SHIFT)
            srtw, _ = plsc.sort_key_val(win, win)
            rank, end = _ranks(srtw, iota)
            plsc.addupdate_scatter(hist, [srtw], rank + 1, mask=end)
            return carry2
        lax.fori_loop(0, NVR, _p1_vec, 0)
        return carry
    lax.fori_loop(0, S, _p1_chunk, 0)

    tot = jnp.int32(0)
    for i in range(8):
        h = hist[pl.ds(i * 16, 16)]
        r16 = jnp.bitwise_and(h + 15, jnp.int32(-16))
        cs = plsc.cumsum(r16)
        seg[pl.ds(i * 16, 16)] = cs - r16 + tot
        base[pl.ds(i * 16, 16)] = cs - r16 + tot
        tot = tot + jnp.sum(r16)

    def _p2_chunk(ci, carry):
        pltpu.sync_copy(tok_hbm.at[pl.ds(ci * B + b0, BT)], idxb[0])

        def _p2_vec(i, carry2):
            t = idxb[0][pl.ds(i * 16, 16)]
            win = jnp.right_shift(t, WIN_SHIFT)
            off = jnp.bitwise_and(t, OFF_MASK)
            pos = ci * CH + i * 16 + iota
            val = jnp.bitwise_or(off, jnp.left_shift(pos, 16))
            srtw, srtv = plsc.sort_key_val(win, val)
            rank, end = _ranks(srtw, iota)
            bs = plsc.load_gather(base, [srtw])
            slot = bs + rank
            plsc.store_scatter(bkt, [slot], srtv)
            plsc.store_scatter(base, [srtw], slot + 1, mask=end)
            return carry2
        lax.fori_loop(0, NVR, _p2_vec, 0)
        return carry
    lax.fori_loop(0, S, _p2_chunk, 0)

    def _sweep(g, carry0):
        rbase = pl.multiple_of(c * 32 + lax.div(g, 2) * 8, 8)
        rsub = lax.rem(g, 2) * 4

        def _issue_hbm(nw, rbase=rbase):
            @pl.when(nw < NWIN - 1)
            def _():
                pltpu.async_copy(
                    wt_hbm.at[pl.ds(rbase, 8), pl.ds(nw * W + s * CSPLIT,
                                                     CSPLIT)],
                    bounce, sem_hbm)

            @pl.when(nw == NWIN - 1)
            def _():
                @pl.when(s == 0)
                def _():
                    pltpu.async_copy(
                        wt_hbm.at[pl.ds(rbase, 8), pl.ds(V_MAIN, CSPLIT)],
                        bounce, sem_hbm)

                @pl.when(s == 1)
                def _():
                    pltpu.async_copy(
                        wtail_hbm.at[pl.ds(rbase, 8)],
                        bounce.at[:, pl.ds(0, 128)], sem_hbm)

        def _wait_hbm(nw, rbase=rbase):
            @pl.when(nw < NWIN - 1)
            def _():
                pltpu.make_async_copy(
                    wt_hbm.at[pl.ds(rbase, 8), pl.ds(nw * W + s * CSPLIT,
                                                     CSPLIT)],
                    bounce, sem_hbm).wait()

            @pl.when(nw == NWIN - 1)
            def _():
                @pl.when(s == 0)
                def _():
                    pltpu.make_async_copy(
                        wt_hbm.at[pl.ds(rbase, 8), pl.ds(V_MAIN, CSPLIT)],
                        bounce, sem_hbm).wait()

                @pl.when(s == 1)
                def _():
                    pltpu.make_async_copy(
                        wtail_hbm.at[pl.ds(rbase, 8)],
                        bounce.at[:, pl.ds(0, 128)], sem_hbm).wait()

        def _spread(nw, nbrow):
            @pl.when(nw < NWIN - 1)
            def _():
                for r in range(8):
                    pltpu.async_copy(
                        bounce.at[r],
                        spflat.at[pl.ds((nbrow + r) * W + s * CSPLIT,
                                        CSPLIT)],
                        sem_spread)

            @pl.when(nw == NWIN - 1)
            def _():
                @pl.when(s == 0)
                def _():
                    for r in range(8):
                        pltpu.async_copy(
                            bounce.at[r],
                            spflat.at[pl.ds((nbrow + r) * W, CSPLIT)],
                            sem_spread)

                @pl.when(s == 1)
                def _():
                    for r in range(8):
                        pltpu.async_copy(
                            bounce.at[r, pl.ds(0, 128)],
                            spflat.at[pl.ds((nbrow + r) * W + CSPLIT, 128)],
                            sem_spread)

        def _wait_spread(nw, nbrow):
            @pl.when(nw < NWIN - 1)
            def _():
                for r in range(8):
                    pltpu.make_async_copy(
                        bounce.at[r],
                        spflat.at[pl.ds((nbrow + r) * W + s * CSPLIT,
                                        CSPLIT)],
                        sem_spread).wait()

            @pl.when(nw == NWIN - 1)
            def _():
                @pl.when(s == 0)
                def _():
                    for r in range(8):
                        pltpu.make_async_copy(
                            bounce.at[r],
                            spflat.at[pl.ds((nbrow + r) * W, CSPLIT)],
                            sem_spread).wait()

                @pl.when(s == 1)
                def _():
                    for r in range(8):
                        pltpu.make_async_copy(
                            bounce.at[r, pl.ds(0, 128)],
                            spflat.at[pl.ds((nbrow + r) * W + CSPLIT, 128)],
                            sem_spread).wait()

        z = jnp.int32(0)
        _issue_hbm(z)
        _wait_hbm(z)
        _spread(z, z)
        _wait_spread(z, z)
        plsc.subcore_barrier()

        def _consume_chunks(lo, hi, s0, n, brow):
            def _chunk(k, carry2):
                c0 = s0 + k * CH
                rowbase = (brow + rsub) * W

                def _idx_vec(i, carry3):
                    v = bkt[pl.ds(c0 + i * 16, 16)]
                    off = jnp.bitwise_and(v, OFF_MASK)
                    for dd in range(G):
                        idxb[dd][pl.ds(i * 16, 16)] = off + (rowbase + dd * W)
                    return carry3
                lax.fori_loop(0, NVR, _idx_vec, 0)

                copies = [
                    pltpu.async_copy(spflat.at[idxb[dd]], gath[dd],
                                     sem_gath)
                    for dd in range(G)
                ]
                for cp in copies:
                    cp.wait()

                def _perm_vec(i, carry3):
                    v = bkt[pl.ds(c0 + i * 16, 16)]
                    pos = lax.shift_right_logical(v, 16)
                    in_seg = (k * CH + i * 16 + iota) < n
                    m = in_seg & (pos < TOK)
                    for dd in range(G):
                        gv = gath[dd][pl.ds(i * 16, 16)]
                        plsc.store_scatter(outv[dd], [pos], gv, mask=m)
                    return carry3
                lax.fori_loop(0, NVR, _perm_vec, 0)
                return carry2
            lax.fori_loop(lo, hi, _chunk, 0)

        def _win_body(w, carry):
            brow = lax.rem(w, 2) * 8
            nbrow = lax.rem(w + 1, 2) * 8
            _issue_hbm(w + 1)

            vseg = seg[pl.ds(w, 16)]
            s0 = vseg[0]
            n = vseg[1] - s0
            nch = lax.div(n + (CH - 1), jnp.int32(CH))
            nch_half = lax.div(nch, jnp.int32(2))

            _consume_chunks(jnp.int32(0), nch_half, s0, n, brow)
            _wait_hbm(w + 1)
            _spread(w + 1, nbrow)
            _consume_chunks(nch_half, nch, s0, n, brow)
            _wait_spread(w + 1, nbrow)
            plsc.subcore_barrier()
            return carry
        lax.fori_loop(0, NWIN, _win_body, 0)

        flush_handles = []
        for dd in range(G):
            for si in range(S):
                dst = out_hbm.at[pl.ds(
                    (si * D + c * 32 + g * G + dd) * B + b0, BT)]
                flush_handles.append(pltpu.async_copy(
                    outv[dd].at[pl.ds(si * BT, BT)], dst, sem_flush))
        for h in flush_handles:
            h.wait()
        return carry0

    lax.fori_loop(0, NSWEEP, _sweep, 0)


def kernel(token_ids, weight):
    tok_flat = token_ids.astype(jnp.int32).T.reshape(-1)
    wt = weight.T
    wtail = jnp.pad(wt[:, V_CUT:], ((0, 0), (0, 64)))
    out_flat = _embed_kernel(tok_flat, wt, wtail)
    return out_flat.reshape(S, D, B).transpose(2, 0, 1)


# speedup vs baseline: 3.3054x; 3.3054x over previous
import functools

import jax
import jax.numpy as jnp
from jax import lax
from jax.experimental import pallas as pl
from jax.experimental.pallas import tpu as pltpu
from jax.experimental.pallas import tpu_sc as plsc

B = 16384
S = 20
D = 64
V = 1000000
V_MAIN = 999424
V_CUT = 999936

NT = 16
BT = B // NT
TOK = S * BT

W = 8192
WIN_SHIFT = 13
OFF_MASK = W - 1
NWIN = 123
CSPLIT = W // NT

G = 4
NSWEEP = 8
CH = 128
NVR = CH // 16
PVR = BT // 16
BKT_CAP = TOK + NWIN * 15 + CH + 16
SENT = 0x7FFFFFFF

_mesh = plsc.VectorSubcoreMesh(core_axis_name="c", subcore_axis_name="s")


def _ranks(srtw, iota):
    nxt = jnp.take_along_axis(srtw, jnp.minimum(iota + 1, 15), axis=0,
                              mode="promise_in_bounds")
    prv = jnp.take_along_axis(srtw, jnp.maximum(iota - 1, 0), axis=0,
                              mode="promise_in_bounds")
    end = (srtw != nxt) | (iota == 15)
    newg = (srtw != prv) | (iota == 0)
    start = plsc.cummax(jnp.where(newg, iota, 0))
    rank = iota - start
    return rank, end


@functools.partial(
    pl.kernel,
    mesh=_mesh,
    out_type=jax.ShapeDtypeStruct((S * D * B,), jnp.float32),
    scratch_types=[
        pltpu.VMEM((BKT_CAP,), jnp.int32),
        pltpu.VMEM((TOK,), jnp.float32),
        pltpu.VMEM((TOK,), jnp.float32),
        pltpu.VMEM((TOK,), jnp.float32),
        pltpu.VMEM((TOK,), jnp.float32),
        pltpu.VMEM((BT,), jnp.int32),
        pltpu.VMEM((CH,), jnp.int32),
        pltpu.VMEM((CH,), jnp.int32),
        pltpu.VMEM((CH,), jnp.int32),
        pltpu.VMEM((CH,), jnp.int32),
        pltpu.VMEM((CH,), jnp.float32),
        pltpu.VMEM((CH,), jnp.float32),
        pltpu.VMEM((CH,), jnp.float32),
        pltpu.VMEM((CH,), jnp.float32),
        pltpu.VMEM((8, CSPLIT), jnp.float32),
        pltpu.VMEM((128,), jnp.int32),
        pltpu.VMEM((144,), jnp.int32),
        pltpu.VMEM((128,), jnp.int32),
        pltpu.VMEM_SHARED((16 * W,), jnp.float32),
        pltpu.SemaphoreType.DMA,
        pltpu.SemaphoreType.DMA,
        pltpu.SemaphoreType.DMA,
        pltpu.SemaphoreType.DMA,
    ],
    compiler_params=pltpu.CompilerParams(needs_layout_passes=False),
)
def _embed_kernel(tok_hbm, wt_hbm, wtail_hbm, out_hbm,
                  bkt, outv0, outv1, outv2, outv3,
                  tokp, idxb0, idxb1, idxb2, idxb3,
                  gath0, gath1, gath2, gath3,
                  bounce, hist, seg, base, spflat,
                  sem_hbm, sem_spread, sem_gath, sem_flush):
    outv = [outv0, outv1, outv2, outv3]
    idxb = [idxb0, idxb1, idxb2, idxb3]
    gath = [gath0, gath1, gath2, gath3]
    c = lax.axis_index("c")
    s = lax.axis_index("s")
    b0 = s * BT
    iota = lax.iota(jnp.int32, 16)
    zero16 = jnp.zeros((16,), jnp.int32)
    sent16 = jnp.full((16,), SENT, jnp.int32)

    for i in range(8):
        hist[pl.ds(i * 16, 16)] = zero16

    def _init_bkt(i, carry):
        bkt[pl.ds(i * 16, 16)] = sent16
        return carry
    lax.fori_loop(0, BKT_CAP // 16, _init_bkt, 0)

    def _p1_chunk(ci, carry):
        pltpu.sync_copy(tok_hbm.at[pl.ds(ci * B + b0, BT)], tokp)

        def _p1_vec(i, carry2):
            t = tokp[pl.ds(i * 16, 16)]
            win = jnp.right_shift(t, WIN_SHIFT)
            srtw, _ = plsc.sort_key_val(win, win)
            rank, end = _ranks(srtw, iota)
            plsc.addupdate_scatter(hist, [srtw], rank + 1, mask=end)
            return carry2
        lax.fori_loop(0, PVR, _p1_vec, 0)
        return carry
    lax.fori_loop(0, S, _p1_chunk, 0)

    tot = jnp.int32(0)
    for i in range(8):
        h = hist[pl.ds(i * 16, 16)]
        r16 = jnp.bitwise_and(h + 15, jnp.int32(-16))
        cs = plsc.cumsum(r16)
        seg[pl.ds(i * 16, 16)] = cs - r16 + tot
        base[pl.ds(i * 16, 16)] = cs - r16 + tot
        tot = tot + jnp.sum(r16)

    def _p2_chunk(ci, carry):
        pltpu.sync_copy(tok_hbm.at[pl.ds(ci * B + b0, BT)], tokp)

        def _p2_vec(i, carry2):
            t = tokp[pl.ds(i * 16, 16)]
            win = jnp.right_shift(t, WIN_SHIFT)
            off = jnp.bitwise_and(t, OFF_MASK)
            pos = ci * BT + i * 16 + iota
            val = jnp.bitwise_or(off, jnp.left_shift(pos, 16))
            srtw, srtv = plsc.sort_key_val(win, val)
            rank, end = _ranks(srtw, iota)
            bs = plsc.load_gather(base, [srtw])
            slot = bs + rank
            plsc.store_scatter(bkt, [slot], srtv)
            plsc.store_scatter(base, [srtw], slot + 1, mask=end)
            return carry2
        lax.fori_loop(0, PVR, _p2_vec, 0)
        return carry
    lax.fori_loop(0, S, _p2_chunk, 0)

    def _sweep(g, carry0):
        rbase = pl.multiple_of(c * 32 + lax.div(g, 2) * 8, 8)
        rsub = lax.rem(g, 2) * 4

        def _issue_hbm(nw, rbase=rbase):
            @pl.when(nw < NWIN - 1)
            def _():
                pltpu.async_copy(
                    wt_hbm.at[pl.ds(rbase, 8), pl.ds(nw * W + s * CSPLIT,
                                                     CSPLIT)],
                    bounce, sem_hbm)

            @pl.when(nw == NWIN - 1)
            def _():
                @pl.when(s == 0)
                def _():
                    pltpu.async_copy(
                        wt_hbm.at[pl.ds(rbase, 8), pl.ds(V_MAIN, CSPLIT)],
                        bounce, sem_hbm)

                @pl.when(s == 1)
                def _():
                    pltpu.async_copy(
                        wtail_hbm.at[pl.ds(rbase, 8)],
                        bounce.at[:, pl.ds(0, 128)], sem_hbm)

        def _wait_hbm(nw, rbase=rbase):
            @pl.when(nw < NWIN - 1)
            def _():
                pltpu.make_async_copy(
                    wt_hbm.at[pl.ds(rbase, 8), pl.ds(nw * W + s * CSPLIT,
                                                     CSPLIT)],
                    bounce, sem_hbm).wait()

            @pl.when(nw == NWIN - 1)
            def _():
                @pl.when(s == 0)
                def _():
                    pltpu.make_async_copy(
                        wt_hbm.at[pl.ds(rbase, 8), pl.ds(V_MAIN, CSPLIT)],
                        bounce, sem_hbm).wait()

                @pl.when(s == 1)
                def _():
                    pltpu.make_async_copy(
                        wtail_hbm.at[pl.ds(rbase, 8)],
                        bounce.at[:, pl.ds(0, 128)], sem_hbm).wait()

        def _spread(nw, nbrow):
            @pl.when(nw < NWIN - 1)
            def _():
                for r in range(8):
                    pltpu.async_copy(
                        bounce.at[r],
                        spflat.at[pl.ds((nbrow + r) * W + s * CSPLIT,
                                        CSPLIT)],
                        sem_spread)

            @pl.when(nw == NWIN - 1)
            def _():
                @pl.when(s == 0)
                def _():
                    for r in range(8):
                        pltpu.async_copy(
                            bounce.at[r],
                            spflat.at[pl.ds((nbrow + r) * W, CSPLIT)],
                            sem_spread)

                @pl.when(s == 1)
                def _():
                    for r in range(8):
                        pltpu.async_copy(
                            bounce.at[r, pl.ds(0, 128)],
                            spflat.at[pl.ds((nbrow + r) * W + CSPLIT, 128)],
                            sem_spread)

        def _wait_spread(nw, nbrow):
            @pl.when(nw < NWIN - 1)
            def _():
                for r in range(8):
                    pltpu.make_async_copy(
                        bounce.at[r],
                        spflat.at[pl.ds((nbrow + r) * W + s * CSPLIT,
                                        CSPLIT)],
                        sem_spread).wait()

            @pl.when(nw == NWIN - 1)
            def _():
                @pl.when(s == 0)
                def _():
                    for r in range(8):
                        pltpu.make_async_copy(
                            bounce.at[r],
                            spflat.at[pl.ds((nbrow + r) * W, CSPLIT)],
                            sem_spread).wait()

                @pl.when(s == 1)
                def _():
                    for r in range(8):
                        pltpu.make_async_copy(
                            bounce.at[r, pl.ds(0, 128)],
                            spflat.at[pl.ds((nbrow + r) * W + CSPLIT, 128)],
                            sem_spread).wait()

        z = jnp.int32(0)
        _issue_hbm(z)
        _wait_hbm(z)
        _spread(z, z)
        _wait_spread(z, z)
        plsc.subcore_barrier()

        def _consume_chunks(lo, hi, s0, n, brow):
            def _chunk(k, carry2):
                c0 = s0 + k * CH
                rowbase = (brow + rsub) * W

                def _idx_vec(i, carry3):
                    v = bkt[pl.ds(c0 + i * 16, 16)]
                    off = jnp.bitwise_and(v, OFF_MASK)
                    for dd in range(G):
                        idxb[dd][pl.ds(i * 16, 16)] = off + (rowbase + dd * W)
                    return carry3
                lax.fori_loop(0, NVR, _idx_vec, 0)

                copies = [
                    pltpu.async_copy(spflat.at[idxb[dd]], gath[dd],
                                     sem_gath)
                    for dd in range(G)
                ]
                for cp in copies:
                    cp.wait()

                def _perm_vec(i, carry3):
                    v = bkt[pl.ds(c0 + i * 16, 16)]
                    pos = lax.shift_right_logical(v, 16)
                    in_seg = (k * CH + i * 16 + iota) < n
                    m = in_seg & (pos < TOK)
                    for dd in range(G):
                        gv = gath[dd][pl.ds(i * 16, 16)]
                        plsc.store_scatter(outv[dd], [pos], gv, mask=m)
                    return carry3
                nrem = n - k * CH
                nv = jnp.minimum(jnp.int32(NVR),
                                 lax.div(nrem + 15, jnp.int32(16)))
                lax.fori_loop(0, nv, _perm_vec, 0)
                return carry2
            lax.fori_loop(lo, hi, _chunk, 0)

        def _win_body(w, carry):
            brow = lax.rem(w, 2) * 8
            nbrow = lax.rem(w + 1, 2) * 8
            _issue_hbm(w + 1)

            vseg = seg[pl.ds(w, 16)]
            s0 = vseg[0]
            n = vseg[1] - s0
            nch = lax.div(n + (CH - 1), jnp.int32(CH))
            nch_half = lax.div(nch, jnp.int32(2))

            _consume_chunks(jnp.int32(0), nch_half, s0, n, brow)
            _wait_hbm(w + 1)
            _spread(w + 1, nbrow)
            _consume_chunks(nch_half, nch, s0, n, brow)
            _wait_spread(w + 1, nbrow)
            plsc.subcore_barrier()
            return carry
        lax.fori_loop(0, NWIN, _win_body, 0)

        flush_handles = []
        for dd in range(G):
            for si in range(S):
                dst = out_hbm.at[pl.ds(
                    (si * D + c * 32 + g * G + dd) * B + b0, BT)]
                flush_handles.append(pltpu.async_copy(
                    outv[dd].at[pl.ds(si * BT, BT)], dst, sem_flush))
        for h in flush_handles:
            h.wait()
        return carry0

    lax.fori_loop(0, NSWEEP, _sweep, 0)


def kernel(token_ids, weight):
    tok_flat = token_ids.astype(jnp.int32).T.reshape(-1)
    wt = weight.T
    wtail = jnp.pad(wt[:, V_CUT:], ((0, 0), (0, 64)))
    out_flat = _embed_kernel(tok_flat, wt, wtail)
    return out_flat.reshape(S, D, B).transpose(2, 0, 1)


# speedup vs baseline: 3.9775x; 1.2034x over previous
import functools

import jax
import jax.numpy as jnp
from jax import lax
from jax.experimental import pallas as pl
from jax.experimental.pallas import tpu as pltpu
from jax.experimental.pallas import tpu_sc as plsc

B = 16384
S = 20
D = 64
V = 1000000
V_MAIN = 999424
V_CUT = 999936

NT = 16
BT = B // NT
TOK = S * BT

W = 16384
WIN_SHIFT = 14
OFF_MASK = W - 1
NWIN = 62
CSPLIT = W // NT
TAILW = 512

G = 4
NSWEEP = 8
CH = 128
NVR = CH // 16
PVR = BT // 16
BKT_CAP = TOK + NWIN * 15 + CH + 16
SENT = 0x7FFFFFFF

_mesh = plsc.VectorSubcoreMesh(core_axis_name="c", subcore_axis_name="s")


def _ranks(srtw, iota):
    nxt = jnp.take_along_axis(srtw, jnp.minimum(iota + 1, 15), axis=0,
                              mode="promise_in_bounds")
    prv = jnp.take_along_axis(srtw, jnp.maximum(iota - 1, 0), axis=0,
                              mode="promise_in_bounds")
    end = (srtw != nxt) | (iota == 15)
    newg = (srtw != prv) | (iota == 0)
    start = plsc.cummax(jnp.where(newg, iota, 0))
    rank = iota - start
    return rank, end


@functools.partial(
    pl.kernel,
    mesh=_mesh,
    out_type=jax.ShapeDtypeStruct((S * D * B,), jnp.float32),
    scratch_types=[
        pltpu.VMEM((BKT_CAP,), jnp.int32),
        pltpu.VMEM((TOK,), jnp.float32),
        pltpu.VMEM((TOK,), jnp.float32),
        pltpu.VMEM((TOK,), jnp.float32),
        pltpu.VMEM((TOK,), jnp.float32),
        pltpu.VMEM((BT,), jnp.int32),
        pltpu.VMEM((CH,), jnp.int32),
        pltpu.VMEM((CH,), jnp.int32),
        pltpu.VMEM((CH,), jnp.int32),
        pltpu.VMEM((CH,), jnp.int32),
        pltpu.VMEM((CH,), jnp.float32),
        pltpu.VMEM((CH,), jnp.float32),
        pltpu.VMEM((CH,), jnp.float32),
        pltpu.VMEM((CH,), jnp.float32),
        pltpu.VMEM((8, CSPLIT), jnp.float32),
        pltpu.VMEM((128,), jnp.int32),
        pltpu.VMEM((144,), jnp.int32),
        pltpu.VMEM((128,), jnp.int32),
        pltpu.VMEM_SHARED((16 * W,), jnp.float32),
        pltpu.SemaphoreType.DMA,
        pltpu.SemaphoreType.DMA,
        pltpu.SemaphoreType.DMA,
        pltpu.SemaphoreType.DMA,
    ],
    compiler_params=pltpu.CompilerParams(needs_layout_passes=False),
)
def _embed_kernel(tok_hbm, wt_hbm, wtail_hbm, out_hbm,
                  bkt, outv0, outv1, outv2, outv3,
                  tokp, idxb0, idxb1, idxb2, idxb3,
                  gath0, gath1, gath2, gath3,
                  bounce, hist, seg, base, spflat,
                  sem_hbm, sem_spread, sem_gath, sem_flush):
    outv = [outv0, outv1, outv2, outv3]
    idxb = [idxb0, idxb1, idxb2, idxb3]
    gath = [gath0, gath1, gath2, gath3]
    c = lax.axis_index("c")
    s = lax.axis_index("s")
    b0 = s * BT
    iota = lax.iota(jnp.int32, 16)
    zero16 = jnp.zeros((16,), jnp.int32)
    sent16 = jnp.full((16,), SENT, jnp.int32)

    for i in range(8):
        hist[pl.ds(i * 16, 16)] = zero16

    def _init_bkt(i, carry):
        bkt[pl.ds(i * 16, 16)] = sent16
        return carry
    lax.fori_loop(0, BKT_CAP // 16, _init_bkt, 0)

    def _p1_chunk(ci, carry):
        pltpu.sync_copy(tok_hbm.at[pl.ds(ci * B + b0, BT)], tokp)

        def _p1_vec(i, carry2):
            t = tokp[pl.ds(i * 16, 16)]
            win = jnp.right_shift(t, WIN_SHIFT)
            srtw, _ = plsc.sort_key_val(win, win)
            rank, end = _ranks(srtw, iota)
            plsc.addupdate_scatter(hist, [srtw], rank + 1, mask=end)
            return carry2
        lax.fori_loop(0, PVR, _p1_vec, 0)
        return carry
    lax.fori_loop(0, S, _p1_chunk, 0)

    tot = jnp.int32(0)
    for i in range(8):
        h = hist[pl.ds(i * 16, 16)]
        r16 = jnp.bitwise_and(h + 15, jnp.int32(-16))
        cs = plsc.cumsum(r16)
        seg[pl.ds(i * 16, 16)] = cs - r16 + tot
        base[pl.ds(i * 16, 16)] = cs - r16 + tot
        tot = tot + jnp.sum(r16)

    def _p2_chunk(ci, carry):
        pltpu.sync_copy(tok_hbm.at[pl.ds(ci * B + b0, BT)], tokp)

        def _p2_vec(i, carry2):
            t = tokp[pl.ds(i * 16, 16)]
            win = jnp.right_shift(t, WIN_SHIFT)
            off = jnp.bitwise_and(t, OFF_MASK)
            pos = ci * BT + i * 16 + iota
            val = jnp.bitwise_or(off, jnp.left_shift(pos, 16))
            srtw, srtv = plsc.sort_key_val(win, val)
            rank, end = _ranks(srtw, iota)
            bs = plsc.load_gather(base, [srtw])
            slot = bs + rank
            plsc.store_scatter(bkt, [slot], srtv)
            plsc.store_scatter(base, [srtw], slot + 1, mask=end)
            return carry2
        lax.fori_loop(0, PVR, _p2_vec, 0)
        return carry
    lax.fori_loop(0, S, _p2_chunk, 0)

    def _sweep(g, carry0):
        rbase = pl.multiple_of(c * 32 + lax.div(g, 2) * 8, 8)
        rsub = lax.rem(g, 2) * 4

        def _issue_hbm(nw, rbase=rbase):
            @pl.when(nw < NWIN - 1)
            def _():
                pltpu.async_copy(
                    wt_hbm.at[pl.ds(rbase, 8), pl.ds(nw * W + s * CSPLIT,
                                                     CSPLIT)],
                    bounce, sem_hbm)

            @pl.when(nw == NWIN - 1)
            def _():
                @pl.when(s == 0)
                def _():
                    pltpu.async_copy(
                        wt_hbm.at[pl.ds(rbase, 8), pl.ds(V_MAIN, TAILW)],
                        bounce.at[:, pl.ds(0, TAILW)], sem_hbm)

                @pl.when(s == 1)
                def _():
                    pltpu.async_copy(
                        wtail_hbm.at[pl.ds(rbase, 8)],
                        bounce.at[:, pl.ds(0, 128)], sem_hbm)

        def _wait_hbm(nw, rbase=rbase):
            @pl.when(nw < NWIN - 1)
            def _():
                pltpu.make_async_copy(
                    wt_hbm.at[pl.ds(rbase, 8), pl.ds(nw * W + s * CSPLIT,
                                                     CSPLIT)],
                    bounce, sem_hbm).wait()

            @pl.when(nw == NWIN - 1)
            def _():
                @pl.when(s == 0)
                def _():
                    pltpu.make_async_copy(
                        wt_hbm.at[pl.ds(rbase, 8), pl.ds(V_MAIN, TAILW)],
                        bounce.at[:, pl.ds(0, TAILW)], sem_hbm).wait()

                @pl.when(s == 1)
                def _():
                    pltpu.make_async_copy(
                        wtail_hbm.at[pl.ds(rbase, 8)],
                        bounce.at[:, pl.ds(0, 128)], sem_hbm).wait()

        def _spread(nw, nbrow):
            @pl.when(nw < NWIN - 1)
            def _():
                for r in range(8):
                    pltpu.async_copy(
                        bounce.at[r],
                        spflat.at[pl.ds((nbrow + r) * W + s * CSPLIT,
                                        CSPLIT)],
                        sem_spread)

            @pl.when(nw == NWIN - 1)
            def _():
                @pl.when(s == 0)
                def _():
                    for r in range(8):
                        pltpu.async_copy(
                            bounce.at[r, pl.ds(0, TAILW)],
                            spflat.at[pl.ds((nbrow + r) * W, TAILW)],
                            sem_spread)

                @pl.when(s == 1)
                def _():
                    for r in range(8):
                        pltpu.async_copy(
                            bounce.at[r, pl.ds(0, 128)],
                            spflat.at[pl.ds((nbrow + r) * W + TAILW, 128)],
                            sem_spread)

        def _wait_spread(nw, nbrow):
            @pl.when(nw < NWIN - 1)
            def _():
                for r in range(8):
                    pltpu.make_async_copy(
                        bounce.at[r],
                        spflat.at[pl.ds((nbrow + r) * W + s * CSPLIT,
                                        CSPLIT)],
                        sem_spread).wait()

            @pl.when(nw == NWIN - 1)
            def _():
                @pl.when(s == 0)
                def _():
                    for r in range(8):
                        pltpu.make_async_copy(
                            bounce.at[r, pl.ds(0, TAILW)],
                            spflat.at[pl.ds((nbrow + r) * W, TAILW)],
                            sem_spread).wait()

                @pl.when(s == 1)
                def _():
                    for r in range(8):
                        pltpu.make_async_copy(
                            bounce.at[r, pl.ds(0, 128)],
                            spflat.at[pl.ds((nbrow + r) * W + TAILW, 128)],
                            sem_spread).wait()

        z = jnp.int32(0)
        _issue_hbm(z)
        _wait_hbm(z)
        _spread(z, z)
        _wait_spread(z, z)
        plsc.subcore_barrier()

        def _consume_chunks(lo, hi, s0, n, brow):
            def _chunk(k, carry2):
                c0 = s0 + k * CH
                rowbase = (brow + rsub) * W

                def _idx_vec(i, carry3):
                    v = bkt[pl.ds(c0 + i * 16, 16)]
                    off = jnp.bitwise_and(v, OFF_MASK)
                    for dd in range(G):
                        idxb[dd][pl.ds(i * 16, 16)] = off + (rowbase + dd * W)
                    return carry3
                lax.fori_loop(0, NVR, _idx_vec, 0)

                copies = [
                    pltpu.async_copy(spflat.at[idxb[dd]], gath[dd],
                                     sem_gath)
                    for dd in range(G)
                ]
                for cp in copies:
                    cp.wait()

                def _perm_vec(i, carry3):
                    v = bkt[pl.ds(c0 + i * 16, 16)]
                    pos = lax.shift_right_logical(v, 16)
                    in_seg = (k * CH + i * 16 + iota) < n
                    m = in_seg & (pos < TOK)
                    for dd in range(G):
                        gv = gath[dd][pl.ds(i * 16, 16)]
                        plsc.store_scatter(outv[dd], [pos], gv, mask=m)
                    return carry3
                nrem = n - k * CH
                nv = jnp.minimum(jnp.int32(NVR),
                                 lax.div(nrem + 15, jnp.int32(16)))
                lax.fori_loop(0, nv, _perm_vec, 0)
                return carry2
            lax.fori_loop(lo, hi, _chunk, 0)

        def _win_body(w, carry):
            brow = lax.rem(w, 2) * 8
            nbrow = lax.rem(w + 1, 2) * 8
            _issue_hbm(w + 1)

            vseg = seg[pl.ds(w, 16)]
            s0 = vseg[0]
            n = vseg[1] - s0
            nch = lax.div(n + (CH - 1), jnp.int32(CH))
            nch_half = lax.div(nch, jnp.int32(2))

            _consume_chunks(jnp.int32(0), nch_half, s0, n, brow)
            _wait_hbm(w + 1)
            _spread(w + 1, nbrow)
            _consume_chunks(nch_half, nch, s0, n, brow)
            _wait_spread(w + 1, nbrow)
            plsc.subcore_barrier()
            return carry
        lax.fori_loop(0, NWIN, _win_body, 0)

        flush_handles = []
        for dd in range(G):
            for si in range(S):
                dst = out_hbm.at[pl.ds(
                    (si * D + c * 32 + g * G + dd) * B + b0, BT)]
                flush_handles.append(pltpu.async_copy(
                    outv[dd].at[pl.ds(si * BT, BT)], dst, sem_flush))
        for h in flush_handles:
            h.wait()
        return carry0

    lax.fori_loop(0, NSWEEP, _sweep, 0)


def kernel(token_ids, weight):
    tok_flat = token_ids.astype(jnp.int32).T.reshape(-1)
    wt = weight.T
    wtail = jnp.pad(wt[:, V_CUT:], ((0, 0), (0, 64)))
    out_flat = _embed_kernel(tok_flat, wt, wtail)
    return out_flat.reshape(S, D, B).transpose(2, 0, 1)


# speedup vs baseline: 4.0614x; 1.0211x over previous
import functools

import jax
import jax.numpy as jnp
from jax import lax
from jax.experimental import pallas as pl
from jax.experimental.pallas import tpu as pltpu
from jax.experimental.pallas import tpu_sc as plsc

B = 16384
S = 20
D = 64
V = 1000000
V_MAIN = 999424
V_CUT = 999936

NT = 16
BT = B // NT
TOK = S * BT

W = 16384
WIN_SHIFT = 14
OFF_MASK = W - 1
NWIN = 62
CSPLIT = W // NT
TAILW = 512

G = 4
NSWEEP = 8
CH = 256
NVR = CH // 16
NPREP = TOK // CH
BKT_CAP = TOK + NWIN * 15 + CH + 16
SENT = 0x7FFFFFFF

_mesh = plsc.VectorSubcoreMesh(core_axis_name="c", subcore_axis_name="s")


def _ranks(srtw, iota):
    nxt = jnp.take_along_axis(srtw, jnp.minimum(iota + 1, 15), axis=0,
                              mode="promise_in_bounds")
    prv = jnp.take_along_axis(srtw, jnp.maximum(iota - 1, 0), axis=0,
                              mode="promise_in_bounds")
    end = (srtw != nxt) | (iota == 15)
    newg = (srtw != prv) | (iota == 0)
    start = plsc.cummax(jnp.where(newg, iota, 0))
    rank = iota - start
    return rank, end


@functools.partial(
    pl.kernel,
    mesh=_mesh,
    out_type=jax.ShapeDtypeStruct((S * D * B,), jnp.float32),
    scratch_types=[
        pltpu.VMEM((BKT_CAP,), jnp.int32),
        pltpu.VMEM((TOK,), jnp.float32),
        pltpu.VMEM((TOK,), jnp.float32),
        pltpu.VMEM((TOK,), jnp.float32),
        pltpu.VMEM((TOK,), jnp.float32),
        pltpu.VMEM((CH,), jnp.int32),
        pltpu.VMEM((CH,), jnp.int32),
        pltpu.VMEM((CH,), jnp.int32),
        pltpu.VMEM((CH,), jnp.int32),
        pltpu.VMEM((CH,), jnp.float32),
        pltpu.VMEM((CH,), jnp.float32),
        pltpu.VMEM((CH,), jnp.float32),
        pltpu.VMEM((CH,), jnp.float32),
        pltpu.VMEM((8, CSPLIT), jnp.float32),
        pltpu.VMEM((128,), jnp.int32),
        pltpu.VMEM((144,), jnp.int32),
        pltpu.VMEM((128,), jnp.int32),
        pltpu.VMEM_SHARED((16 * W,), jnp.float32),
        pltpu.SemaphoreType.DMA,
        pltpu.SemaphoreType.DMA,
        pltpu.SemaphoreType.DMA,
        pltpu.SemaphoreType.DMA,
    ],
    compiler_params=pltpu.CompilerParams(needs_layout_passes=False),
)
def _embed_kernel(tok_hbm, wt_hbm, wtail_hbm, out_hbm,
                  bkt, outv0, outv1, outv2, outv3,
                  idxb0, idxb1, idxb2, idxb3,
                  gath0, gath1, gath2, gath3,
                  bounce, hist, seg, base, spflat,
                  sem_hbm, sem_spread, sem_gath, sem_flush):
    outv = [outv0, outv1, outv2, outv3]
    idxb = [idxb0, idxb1, idxb2, idxb3]
    gath = [gath0, gath1, gath2, gath3]
    c = lax.axis_index("c")
    s = lax.axis_index("s")
    b0 = s * BT
    iota = lax.iota(jnp.int32, 16)
    zero16 = jnp.zeros((16,), jnp.int32)
    sent16 = jnp.full((16,), SENT, jnp.int32)

    for i in range(8):
        hist[pl.ds(i * 16, 16)] = zero16

    def _init_bkt(i, carry):
        bkt[pl.ds(i * 16, 16)] = sent16
        return carry
    lax.fori_loop(0, BKT_CAP // 16, _init_bkt, 0)

    def _p1_chunk(ci, carry):
        srow = lax.div(ci, jnp.int32(BT // CH))
        scol = lax.rem(ci, jnp.int32(BT // CH)) * CH
        pltpu.sync_copy(tok_hbm.at[pl.ds(srow * B + b0 + scol, CH)], idxb[0])

        def _p1_vec(i, carry2):
            t = idxb[0][pl.ds(i * 16, 16)]
            win = jnp.right_shift(t, WIN_SHIFT)
            srtw, _ = plsc.sort_key_val(win, win)
            rank, end = _ranks(srtw, iota)
            plsc.addupdate_scatter(hist, [srtw], rank + 1, mask=end)
            return carry2
        lax.fori_loop(0, NVR, _p1_vec, 0)
        return carry
    lax.fori_loop(0, NPREP, _p1_chunk, 0)

    tot = jnp.int32(0)
    for i in range(8):
        h = hist[pl.ds(i * 16, 16)]
        r16 = jnp.bitwise_and(h + 15, jnp.int32(-16))
        cs = plsc.cumsum(r16)
        seg[pl.ds(i * 16, 16)] = cs - r16 + tot
        base[pl.ds(i * 16, 16)] = cs - r16 + tot
        tot = tot + jnp.sum(r16)

    def _p2_chunk(ci, carry):
        srow = lax.div(ci, jnp.int32(BT // CH))
        scol = lax.rem(ci, jnp.int32(BT // CH)) * CH
        pltpu.sync_copy(tok_hbm.at[pl.ds(srow * B + b0 + scol, CH)], idxb[0])

        def _p2_vec(i, carry2):
            t = idxb[0][pl.ds(i * 16, 16)]
            win = jnp.right_shift(t, WIN_SHIFT)
            off = jnp.bitwise_and(t, OFF_MASK)
            pos = ci * CH + i * 16 + iota
            val = jnp.bitwise_or(off, jnp.left_shift(pos, 16))
            srtw, srtv = plsc.sort_key_val(win, val)
            rank, end = _ranks(srtw, iota)
            bs = plsc.load_gather(base, [srtw])
            slot = bs + rank
            plsc.store_scatter(bkt, [slot], srtv)
            plsc.store_scatter(base, [srtw], slot + 1, mask=end)
            return carry2
        lax.fori_loop(0, NVR, _p2_vec, 0)
        return carry
    lax.fori_loop(0, NPREP, _p2_chunk, 0)

    def _sweep(g, carry0):
        rbase = pl.multiple_of(c * 32 + lax.div(g, 2) * 8, 8)
        rsub = lax.rem(g, 2) * 4

        def _issue_hbm(nw, rbase=rbase):
            @pl.when(nw < NWIN - 1)
            def _():
                pltpu.async_copy(
                    wt_hbm.at[pl.ds(rbase, 8), pl.ds(nw * W + s * CSPLIT,
                                                     CSPLIT)],
                    bounce, sem_hbm)

            @pl.when(nw == NWIN - 1)
            def _():
                @pl.when(s == 0)
                def _():
                    pltpu.async_copy(
                        wt_hbm.at[pl.ds(rbase, 8), pl.ds(V_MAIN, TAILW)],
                        bounce.at[:, pl.ds(0, TAILW)], sem_hbm)

                @pl.when(s == 1)
                def _():
                    pltpu.async_copy(
                        wtail_hbm.at[pl.ds(rbase, 8)],
                        bounce.at[:, pl.ds(0, 128)], sem_hbm)

        def _wait_hbm(nw, rbase=rbase):
            @pl.when(nw < NWIN - 1)
            def _():
                pltpu.make_async_copy(
                    wt_hbm.at[pl.ds(rbase, 8), pl.ds(nw * W + s * CSPLIT,
                                                     CSPLIT)],
                    bounce, sem_hbm).wait()

            @pl.when(nw == NWIN - 1)
            def _():
                @pl.when(s == 0)
                def _():
                    pltpu.make_async_copy(
                        wt_hbm.at[pl.ds(rbase, 8), pl.ds(V_MAIN, TAILW)],
                        bounce.at[:, pl.ds(0, TAILW)], sem_hbm).wait()

                @pl.when(s == 1)
                def _():
                    pltpu.make_async_copy(
                        wtail_hbm.at[pl.ds(rbase, 8)],
                        bounce.at[:, pl.ds(0, 128)], sem_hbm).wait()

        def _spread(nw, nbrow):
            @pl.when(nw < NWIN - 1)
            def _():
                for r in range(8):
                    pltpu.async_copy(
                        bounce.at[r],
                        spflat.at[pl.ds((nbrow + r) * W + s * CSPLIT,
                                        CSPLIT)],
                        sem_spread)

            @pl.when(nw == NWIN - 1)
            def _():
                @pl.when(s == 0)
                def _():
                    for r in range(8):
                        pltpu.async_copy(
                            bounce.at[r, pl.ds(0, TAILW)],
                            spflat.at[pl.ds((nbrow + r) * W, TAILW)],
                            sem_spread)

                @pl.when(s == 1)
                def _():
                    for r in range(8):
                        pltpu.async_copy(
                            bounce.at[r, pl.ds(0, 128)],
                            spflat.at[pl.ds((nbrow + r) * W + TAILW, 128)],
                            sem_spread)

        def _wait_spread(nw, nbrow):
            @pl.when(nw < NWIN - 1)
            def _():
                for r in range(8):
                    pltpu.make_async_copy(
                        bounce.at[r],
                        spflat.at[pl.ds((nbrow + r) * W + s * CSPLIT,
                                        CSPLIT)],
                        sem_spread).wait()

            @pl.when(nw == NWIN - 1)
            def _():
                @pl.when(s == 0)
                def _():
                    for r in range(8):
                        pltpu.make_async_copy(
                            bounce.at[r, pl.ds(0, TAILW)],
                            spflat.at[pl.ds((nbrow + r) * W, TAILW)],
                            sem_spread).wait()

                @pl.when(s == 1)
                def _():
                    for r in range(8):
                        pltpu.make_async_copy(
                            bounce.at[r, pl.ds(0, 128)],
                            spflat.at[pl.ds((nbrow + r) * W + TAILW, 128)],
                            sem_spread).wait()

        z = jnp.int32(0)
        _issue_hbm(z)
        _wait_hbm(z)
        _spread(z, z)
        _wait_spread(z, z)
        plsc.subcore_barrier()

        def _consume_chunks(lo, hi, s0, n, brow):
            def _chunk(k, carry2):
                c0 = s0 + k * CH
                rowbase = (brow + rsub) * W

                def _idx_vec(i, carry3):
                    v = bkt[pl.ds(c0 + i * 16, 16)]
                    off = jnp.bitwise_and(v, OFF_MASK)
                    for dd in range(G):
                        idxb[dd][pl.ds(i * 16, 16)] = off + (rowbase + dd * W)
                    return carry3
                lax.fori_loop(0, NVR, _idx_vec, 0)

                copies = [
                    pltpu.async_copy(spflat.at[idxb[dd]], gath[dd],
                                     sem_gath)
                    for dd in range(G)
                ]
                for cp in copies:
                    cp.wait()

                def _perm_vec(i, carry3):
                    v = bkt[pl.ds(c0 + i * 16, 16)]
                    pos = lax.shift_right_logical(v, 16)
                    in_seg = (k * CH + i * 16 + iota) < n
                    m = in_seg & (pos < TOK)
                    for dd in range(G):
                        gv = gath[dd][pl.ds(i * 16, 16)]
                        plsc.store_scatter(outv[dd], [pos], gv, mask=m)
                    return carry3
                nrem = n - k * CH
                nv = jnp.minimum(jnp.int32(NVR),
                                 lax.div(nrem + 15, jnp.int32(16)))
                lax.fori_loop(0, nv, _perm_vec, 0)
                return carry2
            lax.fori_loop(lo, hi, _chunk, 0)

        def _win_body(w, carry):
            brow = lax.rem(w, 2) * 8
            nbrow = lax.rem(w + 1, 2) * 8
            _issue_hbm(w + 1)

            vseg = seg[pl.ds(w, 16)]
            s0 = vseg[0]
            n = vseg[1] - s0
            nch = lax.div(n + (CH - 1), jnp.int32(CH))
            nch_half = lax.div(nch, jnp.int32(2))

            _consume_chunks(jnp.int32(0), nch_half, s0, n, brow)
            _wait_hbm(w + 1)
            _spread(w + 1, nbrow)
            _consume_chunks(nch_half, nch, s0, n, brow)
            _wait_spread(w + 1, nbrow)
            plsc.subcore_barrier()
            return carry
        lax.fori_loop(0, NWIN, _win_body, 0)

        flush_handles = []
        for dd in range(G):
            for si in range(S):
                dst = out_hbm.at[pl.ds(
                    (si * D + c * 32 + g * G + dd) * B + b0, BT)]
                flush_handles.append(pltpu.async_copy(
                    outv[dd].at[pl.ds(si * BT, BT)], dst, sem_flush))
        for h in flush_handles:
            h.wait()
        return carry0

    lax.fori_loop(0, NSWEEP, _sweep, 0)


def kernel(token_ids, weight):
    tok_flat = token_ids.astype(jnp.int32).T.reshape(-1)
    wt = weight.T
    wtail = jnp.pad(wt[:, V_CUT:], ((0, 0), (0, 64)))
    out_flat = _embed_kernel(tok_flat, wt, wtail)
    return out_flat.reshape(S, D, B).transpose(2, 0, 1)


# speedup vs baseline: 5.8766x; 1.4469x over previous
import functools

import jax
import jax.numpy as jnp
from jax import lax
from jax.experimental import pallas as pl
from jax.experimental.pallas import tpu as pltpu
from jax.experimental.pallas import tpu_sc as plsc

D_MODEL = 64
D_PAD = 128
B_TOTAL = 16384 * 20
NUM_WORKERS = 32
CHUNK = 128
K = 8
ROWS_PER_W = B_TOTAL // (NUM_WORKERS * CHUNK)
NUM_BLOCKS = ROWS_PER_W // K

_mesh = plsc.VectorSubcoreMesh(core_axis_name="c", subcore_axis_name="s")


@functools.partial(
    pl.kernel,
    mesh=_mesh,
    out_type=jax.ShapeDtypeStruct((B_TOTAL, D_PAD), jnp.float32),
    scratch_types=[
        pltpu.VMEM((K, CHUNK), jnp.int32),
        pltpu.VMEM((2, CHUNK, D_PAD), jnp.float32),
        pltpu.SemaphoreType.DMA,
        pltpu.SemaphoreType.DMA,
    ],
)
def _gather_kernel(idx_hbm, table_hbm, out_hbm, idx_v, rows_v, sem_g, sem_s):
    wid = lax.axis_index("s") * 2 + lax.axis_index("c")
    base_row = wid * ROWS_PER_W

    def _gather(h, buf, row):
        return pltpu.async_copy(table_hbm.at[idx_v.at[h]], rows_v.at[buf],
                                sem_g)

    def _store(h, buf, row):
        return pltpu.async_copy(
            rows_v.at[buf],
            out_hbm.at[pl.ds((row + h) * CHUNK, CHUNK)], sem_s)

    def body(blk, carry):
        row = base_row + blk * K
        pltpu.sync_copy(idx_hbm.at[pl.ds(row, K)], idx_v)
        g = [None] * K
        st = [None] * K
        g[0] = _gather(0, 0, row)
        for h in range(K):
            g[h].wait()
            if h + 1 < K:
                if h >= 1:
                    st[h - 1].wait()
                g[h + 1] = _gather(h + 1, (h + 1) % 2, row)
            st[h] = _store(h, h % 2, row)
        st[K - 2].wait()
        st[K - 1].wait()
        return carry

    lax.fori_loop(0, NUM_BLOCKS, body, 0)


def kernel(token_ids, weight):
    idx = token_ids.reshape(B_TOTAL // CHUNK, CHUNK).astype(jnp.int32)
    wp = jnp.pad(weight, ((0, 0), (0, D_PAD - D_MODEL)))
    out = _gather_kernel(idx, wp)
    return out[:, :D_MODEL].reshape(token_ids.shape + (D_MODEL,))


# speedup vs baseline: 7.4212x; 1.2628x over previous
import functools

import jax
import jax.numpy as jnp
from jax import lax
from jax.experimental import pallas as pl
from jax.experimental.pallas import tpu as pltpu
from jax.experimental.pallas import tpu_sc as plsc

D_MODEL = 64
D_PAD = 128
B_TOTAL = 16384 * 20
NUM_WORKERS = 32
CHUNK = 128
K = 8
ROWS_PER_W = B_TOTAL // (NUM_WORKERS * CHUNK)
NUM_BLOCKS = ROWS_PER_W // K

_mesh = plsc.VectorSubcoreMesh(core_axis_name="c", subcore_axis_name="s")


@functools.partial(
    pl.kernel,
    mesh=_mesh,
    out_type=jax.ShapeDtypeStruct((B_TOTAL, D_PAD), jnp.float32),
    scratch_types=[
        pltpu.VMEM((K, CHUNK), jnp.int32),
        pltpu.VMEM((6, CHUNK, D_PAD), jnp.float32),
        pltpu.SemaphoreType.DMA,
        pltpu.SemaphoreType.DMA,
    ],
)
def _gather_kernel(idx_hbm, table_hbm, out_hbm, idx_v, rows_v, sem_g, sem_s):
    wid = lax.axis_index("s") * 2 + lax.axis_index("c")
    base_row = wid * ROWS_PER_W

    def _gather(h, buf, row):
        return pltpu.async_copy(table_hbm.at[idx_v.at[h]], rows_v.at[buf],
                                sem_g)

    def _store(h, buf, row):
        return pltpu.async_copy(
            rows_v.at[buf],
            out_hbm.at[pl.ds((row + h) * CHUNK, CHUNK)], sem_s)

    def body(blk, carry):
        row = base_row + blk * K
        pltpu.sync_copy(idx_hbm.at[pl.ds(row, K)], idx_v)
        g = [None] * K
        st = [None] * K
        for h in range(4):
            g[h] = _gather(h, h % 6, row)
        for h in range(K):
            g[h].wait()
            if h + 4 < K:
                if h >= 2:
                    st[h - 2].wait()
                g[h + 4] = _gather(h + 4, (h + 4) % 6, row)
            st[h] = _store(h, h % 6, row)
        st[K - 2].wait()
        st[K - 1].wait()
        return carry

    lax.fori_loop(0, NUM_BLOCKS, body, 0)


def kernel(token_ids, weight):
    idx = token_ids.reshape(B_TOTAL // CHUNK, CHUNK).astype(jnp.int32)
    wp = jnp.pad(weight, ((0, 0), (0, D_PAD - D_MODEL)))
    out = _gather_kernel(idx, wp)
    return out[:, :D_MODEL].reshape(token_ids.shape + (D_MODEL,))
